# speedup vs baseline: 5.0642x; 5.0642x over previous
import jax
import jax.numpy as jnp
from jax.experimental import pallas as pl


_BLK = 512


def _add_kernel(x_ref, p_ref, o_ref):
    o_ref[...] = x_ref[...] + p_ref[...][:, None, :]


def kernel(x, pos_embed):
    S, B, D = x.shape
    blk = _BLK if S % _BLK == 0 else S
    return pl.pallas_call(
        _add_kernel,
        grid=(S // blk,),
        in_specs=[
            pl.BlockSpec((blk, B, D), lambda i: (i, 0, 0)),
            pl.BlockSpec((blk, D), lambda i: (i, 0)),
        ],
        out_specs=pl.BlockSpec((blk, B, D), lambda i: (i, 0, 0)),
        out_shape=jax.ShapeDtypeStruct((S, B, D), x.dtype),
    )(x, pos_embed[:S])


# speedup vs baseline: 5.1050x; 1.0080x over previous
import jax
import jax.numpy as jnp
from jax.experimental import pallas as pl


_BLK = 1024


def _add_kernel(x_ref, p_ref, o_ref):
    o_ref[...] = x_ref[...] + p_ref[...][:, None, :]


def kernel(x, pos_embed):
    S, B, D = x.shape
    blk = _BLK if S % _BLK == 0 else S
    return pl.pallas_call(
        _add_kernel,
        grid=(S // blk,),
        in_specs=[
            pl.BlockSpec((blk, B, D), lambda i: (i, 0, 0)),
            pl.BlockSpec((blk, D), lambda i: (i, 0)),
        ],
        out_specs=pl.BlockSpec((blk, B, D), lambda i: (i, 0, 0)),
        out_shape=jax.ShapeDtypeStruct((S, B, D), x.dtype),
    )(x, pos_embed[:S])
